# baseline (speedup 1.0000x reference)
import functools

import jax
import jax.numpy as jnp
from jax.experimental import pallas as pl
from jax.experimental.pallas import tpu as pltpu

_MARGIN = 0.3
_N = 4096
_D = 512
_BLK = 512
_NBLK = _N // _BLK
_STEPS = _NBLK * _NBLK


def _masked_sum(sim, ti, tj):
    same = ti[:, None] == tj[None, :]
    pos = jnp.maximum(1.0 - sim, 0.0)
    neg = jnp.where(sim > _MARGIN, sim, 0.0)
    return jnp.sum(jnp.where(same, pos, neg)).reshape(1, 1)


def _loss_body(a_ref, b_ref, tip_ref, tjp_ref, out_ref, s0, s1):
    s = pl.program_id(0)

    @pl.when(s == 0)
    def _init():
        out_ref[...] = jnp.zeros((1, 1), jnp.float32)

    def _dot():
        return jax.lax.dot_general(
            a_ref[...],
            b_ref[...],
            dimension_numbers=(((1,), (1,)), ((), ())),
            preferred_element_type=jnp.float32,
        )

    @pl.when(s % 2 == 0)
    def _even():
        s0[...] = _dot()

        @pl.when(s > 0)
        def _epi():
            out_ref[...] += _masked_sum(s1[...], tip_ref[...], tjp_ref[...])

    @pl.when(s % 2 == 1)
    def _odd():
        s1[...] = _dot()

        @pl.when(s > 0)
        def _epi():
            out_ref[...] += _masked_sum(s0[...], tip_ref[...], tjp_ref[...])

    @pl.when(s == _STEPS)
    def _finish():
        out_ref[...] *= 1.0 / _N


@functools.partial(jax.jit, static_argnames=())
def kernel(inputs, targets):
    t32 = targets.astype(jnp.int32)
    x16 = inputs.astype(jnp.bfloat16)
    total = pl.pallas_call(
        _loss_body,
        grid=(_STEPS + 1,),
        in_specs=[
            pl.BlockSpec((_BLK, _D), lambda s: (jnp.minimum(s // _NBLK, _NBLK - 1), 0)),
            pl.BlockSpec((_BLK, _D), lambda s: (s % _NBLK, 0)),
            pl.BlockSpec((_BLK,), lambda s: (jnp.clip((s - 1) // _NBLK, 0, _NBLK - 1),)),
            pl.BlockSpec((_BLK,), lambda s: ((s - 1) % _NBLK,)),
        ],
        out_specs=pl.BlockSpec((1, 1), lambda s: (0, 0)),
        out_shape=jax.ShapeDtypeStruct((1, 1), jnp.float32),
        scratch_shapes=[
            pltpu.VMEM((_BLK, _BLK), jnp.float32),
            pltpu.VMEM((_BLK, _BLK), jnp.float32),
        ],
    )(x16, x16, t32, t32)
    return total[0, 0]


# speedup vs baseline: 1.7461x; 1.7461x over previous
import functools

import jax
import jax.numpy as jnp
from jax.experimental import pallas as pl

_MARGIN = 0.3
_N = 4096
_D = 512
_BLK = 512
_NBLK = _N // _BLK


def _loss_body(a_ref, b_ref, ti_ref, tj_ref, out_ref):
    s = pl.program_id(0)

    @pl.when(s == 0)
    def _init():
        out_ref[...] = jnp.zeros((1, 1), jnp.float32)

    sim = jax.lax.dot_general(
        a_ref[...],
        b_ref[...],
        dimension_numbers=(((1,), (1,)), ((), ())),
        preferred_element_type=jnp.float32,
    )
    same = ti_ref[...][:, None] == tj_ref[...][None, :]
    pos = jnp.maximum(1.0 - sim, 0.0)
    neg = jnp.where(sim > _MARGIN, sim, 0.0)
    out_ref[...] += jnp.sum(jnp.where(same, pos, neg)).reshape(1, 1)

    @pl.when(s == _NBLK - 1)
    def _finish():
        out_ref[...] *= 1.0 / _N


@functools.partial(jax.jit, static_argnames=())
def kernel(inputs, targets):
    t32 = targets.astype(jnp.int32)
    x16 = inputs.astype(jnp.bfloat16)
    total = pl.pallas_call(
        _loss_body,
        grid=(_NBLK,),
        in_specs=[
            pl.BlockSpec((_BLK, _D), lambda s: (s, 0)),
            pl.BlockSpec((_N, _D), lambda s: (0, 0)),
            pl.BlockSpec((_BLK,), lambda s: (s,)),
            pl.BlockSpec((_N,), lambda s: (0,)),
        ],
        out_specs=pl.BlockSpec((1, 1), lambda s: (0, 0)),
        out_shape=jax.ShapeDtypeStruct((1, 1), jnp.float32),
    )(x16, x16, t32, t32)
    return total[0, 0]


# speedup vs baseline: 2.3133x; 1.3248x over previous
import functools

import jax
import jax.numpy as jnp
from jax.experimental import pallas as pl

_MARGIN = 0.3
_N = 4096
_D = 512
_BLK = 512
_NBLK = _N // _BLK


def _loss_body(a_ref, b_ref, ti_ref, tj_ref, out_ref):
    s = pl.program_id(0)

    @pl.when(s == 0)
    def _init():
        out_ref[...] = jnp.zeros((1, 1), jnp.float32)

    a = a_ref[...]
    ti = ti_ref[...]
    acc = jnp.zeros((1, 1), jnp.float32)
    for j in range(_NBLK):
        b = b_ref[j * _BLK:(j + 1) * _BLK, :]
        tj = tj_ref[j * _BLK:(j + 1) * _BLK]
        sim = jax.lax.dot_general(
            a,
            b,
            dimension_numbers=(((1,), (1,)), ((), ())),
            preferred_element_type=jnp.float32,
        )
        same = ti[:, None] == tj[None, :]
        pos = jnp.maximum(1.0 - sim, 0.0)
        neg = jnp.where(sim > _MARGIN, sim, 0.0)
        acc += jnp.sum(jnp.where(same, pos, neg)).reshape(1, 1)
    out_ref[...] += acc

    @pl.when(s == _NBLK - 1)
    def _finish():
        out_ref[...] *= 1.0 / _N


@functools.partial(jax.jit, static_argnames=())
def kernel(inputs, targets):
    t32 = targets.astype(jnp.int32)
    x16 = inputs.astype(jnp.bfloat16)
    total = pl.pallas_call(
        _loss_body,
        grid=(_NBLK,),
        in_specs=[
            pl.BlockSpec((_BLK, _D), lambda s: (s, 0)),
            pl.BlockSpec((_N, _D), lambda s: (0, 0)),
            pl.BlockSpec((_BLK,), lambda s: (s,)),
            pl.BlockSpec((_N,), lambda s: (0,)),
        ],
        out_specs=pl.BlockSpec((1, 1), lambda s: (0, 0)),
        out_shape=jax.ShapeDtypeStruct((1, 1), jnp.float32),
    )(x16, x16, t32, t32)
    return total[0, 0]


# speedup vs baseline: 3.0698x; 1.3270x over previous
import functools

import jax
import jax.numpy as jnp
from jax.experimental import pallas as pl

_MARGIN = 0.3
_N = 4096
_D = 512
_BLK = 512
_NBLK = _N // _BLK


def _loss_body(a_ref, b_ref, ti_ref, tj_ref, out_ref):
    s = pl.program_id(0)

    @pl.when(s == 0)
    def _init():
        out_ref[...] = jnp.zeros((1, 1), jnp.float32)

    a = a_ref[...]
    ti = ti_ref[...]
    p = pl.program_id(0)
    acc = jnp.zeros((1, 1), jnp.float32)
    for d, w in ((0, 1.0), (1, 2.0), (2, 2.0), (3, 2.0), (4, 1.0)):
        off = ((p + d) % _NBLK) * _BLK
        b = b_ref[pl.ds(off, _BLK), :]
        tj = tj_ref[pl.ds(off, _BLK)]
        sim = jax.lax.dot_general(
            a,
            b,
            dimension_numbers=(((1,), (1,)), ((), ())),
            preferred_element_type=jnp.float32,
        )
        same = ti[:, None] == tj[None, :]
        pos = jnp.maximum(1.0 - sim, 0.0)
        neg = jnp.where(sim > _MARGIN, sim, 0.0)
        acc += w * jnp.sum(jnp.where(same, pos, neg)).reshape(1, 1)
    out_ref[...] += acc

    @pl.when(s == _NBLK - 1)
    def _finish():
        out_ref[...] *= 1.0 / _N


@functools.partial(jax.jit, static_argnames=())
def kernel(inputs, targets):
    t32 = targets.astype(jnp.int32)
    x16 = inputs.astype(jnp.bfloat16)
    total = pl.pallas_call(
        _loss_body,
        grid=(_NBLK,),
        in_specs=[
            pl.BlockSpec((_BLK, _D), lambda s: (s, 0)),
            pl.BlockSpec((_N, _D), lambda s: (0, 0)),
            pl.BlockSpec((_BLK,), lambda s: (s,)),
            pl.BlockSpec((_N,), lambda s: (0,)),
        ],
        out_specs=pl.BlockSpec((1, 1), lambda s: (0, 0)),
        out_shape=jax.ShapeDtypeStruct((1, 1), jnp.float32),
    )(x16, x16, t32, t32)
    return total[0, 0]


# speedup vs baseline: 3.6798x; 1.1987x over previous
import functools

import jax
import jax.numpy as jnp
from jax.experimental import pallas as pl
from jax.experimental.pallas import tpu as pltpu

_MARGIN = 0.3
_N = 4096
_D = 512
_BLK = 512
_NBLK = _N // _BLK


def _loss_body(x_ref, t_ref, out_ref, bx_ref):
    p = pl.program_id(0)

    @pl.when(p == 0)
    def _init():
        out_ref[...] = jnp.zeros((1, 1), jnp.float32)
        bx_ref[...] = x_ref[...].astype(jnp.bfloat16)

    a = bx_ref[pl.ds(p * _BLK, _BLK), :]
    ti = t_ref[pl.ds(p * _BLK, _BLK)]
    acc = jnp.zeros((1, 1), jnp.float32)
    for d, w in ((0, 1.0), (1, 2.0), (2, 2.0), (3, 2.0), (4, 1.0)):
        off = ((p + d) % _NBLK) * _BLK
        b = bx_ref[pl.ds(off, _BLK), :]
        tj = t_ref[pl.ds(off, _BLK)]
        sim = jax.lax.dot_general(
            a,
            b,
            dimension_numbers=(((1,), (1,)), ((), ())),
            preferred_element_type=jnp.float32,
        )
        same = ti[:, None] == tj[None, :]
        pos = jnp.maximum(1.0 - sim, 0.0)
        neg = jnp.where(sim > _MARGIN, sim, 0.0)
        acc += w * jnp.sum(jnp.where(same, pos, neg)).reshape(1, 1)
    out_ref[...] += acc

    @pl.when(p == _NBLK - 1)
    def _finish():
        out_ref[...] *= 1.0 / _N


@functools.partial(jax.jit, static_argnames=())
def kernel(inputs, targets):
    t32 = targets.astype(jnp.int32)
    total = pl.pallas_call(
        _loss_body,
        grid=(_NBLK,),
        in_specs=[
            pl.BlockSpec((_N, _D), lambda s: (0, 0)),
            pl.BlockSpec((_N,), lambda s: (0,)),
        ],
        out_specs=pl.BlockSpec((1, 1), lambda s: (0, 0)),
        out_shape=jax.ShapeDtypeStruct((1, 1), jnp.float32),
        scratch_shapes=[pltpu.VMEM((_N, _D), jnp.bfloat16)],
    )(inputs, t32)
    return total[0, 0]


# speedup vs baseline: 4.1543x; 1.1289x over previous
import functools

import jax
import jax.numpy as jnp
from jax.experimental import pallas as pl
from jax.experimental.pallas import tpu as pltpu

_MARGIN = 0.3
_N = 4096
_D = 512
_BLK = 512
_NBLK = _N // _BLK


def _loss_body(x_ref, t_ref, out_ref, bx_ref):
    p = pl.program_id(0)

    @pl.when(p == 0)
    def _init():
        out_ref[...] = jnp.zeros((1, 1), jnp.float32)
        bx_ref[...] = x_ref[...].astype(jnp.bfloat16)

    acc = jnp.zeros((1, 1), jnp.float32)
    for sub in (0, 1):
        r = 2 * p + sub
        a = bx_ref[pl.ds(r * _BLK, _BLK), :]
        ti = t_ref[pl.ds(r * _BLK, _BLK)]
        for d, w in ((0, 1.0), (1, 2.0), (2, 2.0), (3, 2.0), (4, 1.0)):
            off = ((r + d) % _NBLK) * _BLK
            b = bx_ref[pl.ds(off, _BLK), :]
            tj = t_ref[pl.ds(off, _BLK)]
            sim = jax.lax.dot_general(
                a,
                b,
                dimension_numbers=(((1,), (1,)), ((), ())),
                preferred_element_type=jnp.float32,
            )
            same = ti[:, None] == tj[None, :]
            pos = jnp.maximum(1.0 - sim, 0.0)
            neg = jnp.where(sim > _MARGIN, sim, 0.0)
            acc += w * jnp.sum(jnp.where(same, pos, neg)).reshape(1, 1)
    out_ref[...] += acc

    @pl.when(p == _NBLK // 2 - 1)
    def _finish():
        out_ref[...] *= 1.0 / _N


@functools.partial(jax.jit, static_argnames=())
def kernel(inputs, targets):
    t32 = targets.astype(jnp.int32)
    total = pl.pallas_call(
        _loss_body,
        grid=(_NBLK // 2,),
        in_specs=[
            pl.BlockSpec((_N, _D), lambda s: (0, 0)),
            pl.BlockSpec((_N,), lambda s: (0,)),
        ],
        out_specs=pl.BlockSpec((1, 1), lambda s: (0, 0)),
        out_shape=jax.ShapeDtypeStruct((1, 1), jnp.float32),
        scratch_shapes=[pltpu.VMEM((_N, _D), jnp.bfloat16)],
    )(inputs, t32)
    return total[0, 0]


# speedup vs baseline: 4.4919x; 1.0813x over previous
import functools

import jax
import jax.numpy as jnp
from jax.experimental import pallas as pl
from jax.experimental.pallas import tpu as pltpu

_MARGIN = 0.3
_N = 4096
_D = 512
_BLK = 512
_NBLK = _N // _BLK


def _loss_body(x_ref, t_ref, out_ref, bx_ref):
    p = pl.program_id(0)

    @pl.when(p == 0)
    def _init():
        out_ref[...] = jnp.zeros((1, 1), jnp.float32)
        bx_ref[...] = x_ref[...].astype(jnp.bfloat16)

    acc = jnp.zeros((1, 1), jnp.float32)
    for sub in (0, 1, 2, 3):
        r = 4 * p + sub
        a = bx_ref[pl.ds(r * _BLK, _BLK), :]
        ti = t_ref[pl.ds(r * _BLK, _BLK)]
        for d, w in ((0, 1.0), (1, 2.0), (2, 2.0), (3, 2.0), (4, 1.0)):
            off = ((r + d) % _NBLK) * _BLK
            b = bx_ref[pl.ds(off, _BLK), :]
            tj = t_ref[pl.ds(off, _BLK)]
            sim = jax.lax.dot_general(
                a,
                b,
                dimension_numbers=(((1,), (1,)), ((), ())),
                preferred_element_type=jnp.float32,
            )
            same = ti[:, None] == tj[None, :]
            pos = jnp.maximum(1.0 - sim, 0.0)
            neg = jnp.where(sim > _MARGIN, sim, 0.0)
            acc += w * jnp.sum(jnp.where(same, pos, neg)).reshape(1, 1)
    out_ref[...] += acc

    @pl.when(p == _NBLK // 4 - 1)
    def _finish():
        out_ref[...] *= 1.0 / _N


@functools.partial(jax.jit, static_argnames=())
def kernel(inputs, targets):
    t32 = targets.astype(jnp.int32)
    total = pl.pallas_call(
        _loss_body,
        grid=(_NBLK // 4,),
        in_specs=[
            pl.BlockSpec((_N, _D), lambda s: (0, 0)),
            pl.BlockSpec((_N,), lambda s: (0,)),
        ],
        out_specs=pl.BlockSpec((1, 1), lambda s: (0, 0)),
        out_shape=jax.ShapeDtypeStruct((1, 1), jnp.float32),
        scratch_shapes=[pltpu.VMEM((_N, _D), jnp.bfloat16)],
    )(inputs, t32)
    return total[0, 0]


# speedup vs baseline: 4.6441x; 1.0339x over previous
import functools

import jax
import jax.numpy as jnp
from jax.experimental import pallas as pl
from jax.experimental.pallas import tpu as pltpu

_MARGIN = 0.3
_N = 4096
_D = 512
_BLK = 512
_NBLK = _N // _BLK


def _loss_body(x_ref, t_ref, out_ref, bx_ref):
    bx_ref[...] = x_ref[...].astype(jnp.bfloat16)

    acc = jnp.zeros((1, 1), jnp.float32)
    for r in range(_NBLK):
        a = bx_ref[r * _BLK:(r + 1) * _BLK, :]
        ti = t_ref[r * _BLK:(r + 1) * _BLK]
        for d, w in ((0, 1.0), (1, 2.0), (2, 2.0), (3, 2.0), (4, 1.0)):
            j = (r + d) % _NBLK
            b = bx_ref[j * _BLK:(j + 1) * _BLK, :]
            tj = t_ref[j * _BLK:(j + 1) * _BLK]
            sim = jax.lax.dot_general(
                a,
                b,
                dimension_numbers=(((1,), (1,)), ((), ())),
                preferred_element_type=jnp.float32,
            )
            same = ti[:, None] == tj[None, :]
            pos = jnp.maximum(1.0 - sim, 0.0)
            neg = jnp.where(sim > _MARGIN, sim, 0.0)
            acc += w * jnp.sum(jnp.where(same, pos, neg)).reshape(1, 1)
    out_ref[...] = acc * (1.0 / _N)


@functools.partial(jax.jit, static_argnames=())
def kernel(inputs, targets):
    t32 = targets.astype(jnp.int32)
    total = pl.pallas_call(
        _loss_body,
        grid=(1,),
        in_specs=[
            pl.BlockSpec((_N, _D), lambda s: (0, 0)),
            pl.BlockSpec((_N,), lambda s: (0,)),
        ],
        out_specs=pl.BlockSpec((1, 1), lambda s: (0, 0)),
        out_shape=jax.ShapeDtypeStruct((1, 1), jnp.float32),
        scratch_shapes=[pltpu.VMEM((_N, _D), jnp.bfloat16)],
    )(inputs, t32)
    return total[0, 0]


# speedup vs baseline: 4.9618x; 1.0684x over previous
import functools

import jax
import jax.numpy as jnp
from jax.experimental import pallas as pl
from jax.experimental.pallas import tpu as pltpu

_MARGIN = 0.3
_N = 4096
_D = 512
_BLK = 512
_NBLK = _N // _BLK


def _loss_body(x_ref, t_ref, out_ref, bx_ref):
    bx_ref[...] = x_ref[...].astype(jnp.bfloat16)

    acc = jnp.zeros((1, 1), jnp.float32)
    for r in range(_NBLK):
        a = bx_ref[r * _BLK:(r + 1) * _BLK, :]
        ti = t_ref[r * _BLK:(r + 1) * _BLK]
        for j in range(r, _NBLK):
            w = 1.0 if j == r else 2.0
            b = bx_ref[j * _BLK:(j + 1) * _BLK, :]
            tj = t_ref[j * _BLK:(j + 1) * _BLK]
            sim = jax.lax.dot_general(
                a,
                b,
                dimension_numbers=(((1,), (1,)), ((), ())),
                preferred_element_type=jnp.float32,
            )
            same = ti[:, None] == tj[None, :]
            pos = jnp.maximum(1.0 - sim, 0.0)
            neg = jnp.where(sim > _MARGIN, sim, 0.0)
            acc += w * jnp.sum(jnp.where(same, pos, neg)).reshape(1, 1)
    out_ref[...] = acc * (1.0 / _N)


@functools.partial(jax.jit, static_argnames=())
def kernel(inputs, targets):
    t32 = targets.astype(jnp.int32)
    total = pl.pallas_call(
        _loss_body,
        grid=(1,),
        in_specs=[
            pl.BlockSpec((_N, _D), lambda s: (0, 0)),
            pl.BlockSpec((_N,), lambda s: (0,)),
        ],
        out_specs=pl.BlockSpec((1, 1), lambda s: (0, 0)),
        out_shape=jax.ShapeDtypeStruct((1, 1), jnp.float32),
        scratch_shapes=[pltpu.VMEM((_N, _D), jnp.bfloat16)],
    )(inputs, t32)
    return total[0, 0]


# speedup vs baseline: 5.0354x; 1.0148x over previous
import functools

import jax
import jax.numpy as jnp
from jax.experimental import pallas as pl
from jax.experimental.pallas import tpu as pltpu

_MARGIN = 0.3
_N = 4096
_D = 512
_BLK = 512
_NBLK = _N // _BLK


def _masked(sim, ti, tj):
    same = ti[:, None] == tj[None, :]
    pos = jnp.maximum(1.0 - sim, 0.0)
    neg = jnp.where(sim > _MARGIN, sim, 0.0)
    return jnp.where(same, pos, neg)


def _loss_body(x_ref, t_ref, out_ref, bx_ref):
    bx_ref[...] = x_ref[...].astype(jnp.bfloat16)

    ri = jax.lax.broadcasted_iota(jnp.int32, (128, 128), 0)
    ci = jax.lax.broadcasted_iota(jnp.int32, (128, 128), 1)
    wdiag = jnp.where(ri < ci, 2.0, jnp.where(ri == ci, 1.0, 0.0))

    acc = jnp.zeros((1, 1), jnp.float32)
    for r in range(_NBLK):
        a = bx_ref[r * _BLK:(r + 1) * _BLK, :]
        ti = t_ref[r * _BLK:(r + 1) * _BLK]
        for j in range(r, _NBLK):
            b = bx_ref[j * _BLK:(j + 1) * _BLK, :]
            tj = t_ref[j * _BLK:(j + 1) * _BLK]
            sim = jax.lax.dot_general(
                a,
                b,
                dimension_numbers=(((1,), (1,)), ((), ())),
                preferred_element_type=jnp.float32,
            )
            if j > r:
                acc += 2.0 * jnp.sum(_masked(sim, ti, tj)).reshape(1, 1)
            else:
                for k in range(4):
                    c0, c1 = 128 * k, 128 * (k + 1)
                    tjk = ti[c0:c1]
                    if k > 0:
                        rect = _masked(sim[0:c0, c0:c1], ti[0:c0], tjk)
                        acc += 2.0 * jnp.sum(rect).reshape(1, 1)
                    dblk = _masked(sim[c0:c1, c0:c1], tjk, tjk)
                    acc += jnp.sum(dblk * wdiag).reshape(1, 1)
    out_ref[...] = acc * (1.0 / _N)


@functools.partial(jax.jit, static_argnames=())
def kernel(inputs, targets):
    t32 = targets.astype(jnp.int32)
    total = pl.pallas_call(
        _loss_body,
        grid=(1,),
        in_specs=[
            pl.BlockSpec((_N, _D), lambda s: (0, 0)),
            pl.BlockSpec((_N,), lambda s: (0,)),
        ],
        out_specs=pl.BlockSpec((1, 1), lambda s: (0, 0)),
        out_shape=jax.ShapeDtypeStruct((1, 1), jnp.float32),
        scratch_shapes=[pltpu.VMEM((_N, _D), jnp.bfloat16)],
    )(inputs, t32)
    return total[0, 0]
